# baseline (speedup 1.0000x reference)
import functools

import jax
import jax.numpy as jnp
import numpy as np
from jax import lax
from jax.experimental import pallas as pl
from jax.experimental.pallas import tpu as pltpu
from jax.experimental.pallas import tpu_sc as plsc

_B = 16384
_C = 256
_M = 3
_BR_HI = 1024
_BR_LO = 1024
_B_SC = 3072

_NC = 2
_NS = 16
_NW = _NC * _NS
_NW_ELEMS = _B_SC * _C // _NW

_TINY = np.float32(np.finfo(np.float32).tiny)
_SCALE = np.float32(1.0) - _TINY

_K0 = np.uint32(0)
_K1 = np.uint32(42)
_K2 = _K0 ^ _K1 ^ np.uint32(0x1BD11BDA)
_KS = (_K0, _K1, _K2)
_ROTS = ((13, 15, 26, 6), (17, 29, 16, 24))


def _threefry_bits(j):
    x0 = jnp.zeros_like(j) + _K0
    x1 = j + _K1
    for i in range(5):
        for r in _ROTS[i % 2]:
            x0 = x0 + x1
            x1 = (x1 << np.uint32(r)) | (x1 >> np.uint32(32 - r))
            x1 = x1 ^ x0
        x0 = x0 + _KS[(i + 1) % 3]
        x1 = x1 + _KS[(i + 2) % 3] + np.uint32(i + 1)
    return x0 ^ x1


def _uniform_from_bits(bits):
    fb = (bits >> np.uint32(9)) | np.uint32(0x3F800000)
    f = jax.lax.bitcast_convert_type(fb, jnp.float32) - np.float32(1.0)
    return f + _TINY


def _neglog_u_from_count(j):
    return -jnp.log(_uniform_from_bits(_threefry_bits(j)))


def _sc_noise_body(u0_hbm, u1_hbm, u2_hbm, v0, v1, v2):
    wid = lax.axis_index("s") * _NC + lax.axis_index("c")
    base = wid * _NW_ELEMS
    lane3 = lax.iota(jnp.int32, 16) * 3

    def body(i, carry):
        jv = ((base + i * 16) * 3 + lane3).astype(jnp.uint32)
        sl = pl.ds(i * 16, 16)
        v0[sl] = _uniform_from_bits(_threefry_bits(jv))
        v1[sl] = _uniform_from_bits(_threefry_bits(jv + np.uint32(1)))
        v2[sl] = _uniform_from_bits(_threefry_bits(jv + np.uint32(2)))
        return carry

    lax.fori_loop(0, _NW_ELEMS // 16, body, 0)
    pltpu.sync_copy(v0, u0_hbm.at[pl.ds(base, _NW_ELEMS)])
    pltpu.sync_copy(v1, u1_hbm.at[pl.ds(base, _NW_ELEMS)])
    pltpu.sync_copy(v2, u2_hbm.at[pl.ds(base, _NW_ELEMS)])


_sc_noise = functools.partial(
    pl.kernel,
    out_type=[jax.ShapeDtypeStruct((_B_SC * _C,), jnp.float32)] * 3,
    mesh=plsc.VectorSubcoreMesh(core_axis_name="c", subcore_axis_name="s"),
    scratch_types=[pltpu.VMEM((_NW_ELEMS,), jnp.float32)] * 3,
)(_sc_noise_body)


def _docking(x0_ref, x1_ref, x2_ref, w0_ref, w1_ref, w2_ref, b_ref):
    return (_dock(x0_ref, w0_ref, b_ref, 0),
            _dock(x1_ref, w1_ref, b_ref, 1),
            _dock(x2_ref, w2_ref, b_ref, 2))


def _select(p, l0, l1, l2, d0, d1, d2):
    p0 = p[:, 0:1]
    p1 = p[:, 1:2]
    p2 = p[:, 2:3]
    pick0 = (p0 * l1 >= p1 * l0) & (p0 * l2 >= p2 * l0)
    pick1 = p1 * l2 >= p2 * l1
    return jnp.where(pick0, d0, jnp.where(pick1, d1, d2))


def _probs(sp_ref, av_ref):
    w = sp_ref[...] * av_ref[...]
    return w / jnp.sum(w, axis=1, keepdims=True) + np.float32(1e-20)


def _dock(x_ref, w_ref, b_ref, row):
    return jnp.maximum(
        lax.dot_general(x_ref[...], w_ref[...],
                        (((1,), (1,)), ((), ())),
                        preferred_element_type=jnp.float32)
        + b_ref[row:row + 1, :], 0.0)


def _hi_kernel(x0_ref, x1_ref, x2_ref, sp_ref, av_ref,
               w0_ref, w1_ref, w2_ref, b_ref, out_ref):
    d0, d1, d2 = _docking(x0_ref, x1_ref, x2_ref, w0_ref, w1_ref, w2_ref,
                          b_ref)
    p = _probs(sp_ref, av_ref)
    base = (_B_SC + pl.program_id(0) * _BR_HI) * (_C * _M)
    r = jax.lax.broadcasted_iota(jnp.int32, (_BR_HI, _C), 0)
    c = jax.lax.broadcasted_iota(jnp.int32, (_BR_HI, _C), 1)
    q = base + r * (_C * _M) + c * _M
    l0 = _neglog_u_from_count(q.astype(jnp.uint32))
    l1 = _neglog_u_from_count((q + 1).astype(jnp.uint32))
    l2 = _neglog_u_from_count((q + 2).astype(jnp.uint32))
    out_ref[...] = _select(p, l0, l1, l2, d0, d1, d2)


def _lo_kernel(x0_ref, x1_ref, x2_ref, sp_ref, av_ref,
               w0_ref, w1_ref, w2_ref, b_ref,
               u0_ref, u1_ref, u2_ref, buf_ref, out_ref):
    del buf_ref
    d0, d1, d2 = _docking(x0_ref, x1_ref, x2_ref, w0_ref, w1_ref, w2_ref,
                          b_ref)
    p = _probs(sp_ref, av_ref)
    l0 = -jnp.log(u0_ref[...])
    l1 = -jnp.log(u1_ref[...])
    l2 = -jnp.log(u2_ref[...])
    out_ref[...] = _select(p, l0, l1, l2, d0, d1, d2)


def _row_specs(br, off):
    def rows(i, _off=off):
        return (i + _off, 0)

    return [
        pl.BlockSpec((br, 512), rows),
        pl.BlockSpec((br, 256), rows),
        pl.BlockSpec((br, 128), rows),
        pl.BlockSpec((br, 3), rows),
        pl.BlockSpec((br, 3), rows),
        pl.BlockSpec((256, 512), lambda i: (0, 0)),
        pl.BlockSpec((256, 256), lambda i: (0, 0)),
        pl.BlockSpec((256, 128), lambda i: (0, 0)),
        pl.BlockSpec((3, 256), lambda i: (0, 0)),
    ]


def kernel(input_0, input_1, input_2, availabilities, selection_probabilities,
           W0, b0, W1, b1, W2, b2):
    sp3 = selection_probabilities.astype(jnp.float32)
    av3 = availabilities.astype(jnp.float32)
    bstack = jnp.stack([b0, b1, b2], axis=0)

    u0, u1, u2 = _sc_noise()
    u0 = u0.reshape(_B_SC, _C)
    u1 = u1.reshape(_B_SC, _C)
    u2 = u2.reshape(_B_SC, _C)

    nsc = _B_SC // _BR_HI
    buf = pl.pallas_call(
        _hi_kernel,
        grid=((_B - _B_SC) // _BR_HI,),
        in_specs=_row_specs(_BR_HI, nsc),
        out_specs=pl.BlockSpec((_BR_HI, _C), lambda i, _off=nsc: (i + _off, 0)),
        out_shape=jax.ShapeDtypeStruct((_B, _C), jnp.float32),
        compiler_params=pltpu.CompilerParams(
            dimension_semantics=("parallel",)),
    )(input_0, input_1, input_2, sp3, av3, W0, W1, W2, bstack)

    return pl.pallas_call(
        _lo_kernel,
        grid=(_B_SC // _BR_LO,),
        in_specs=_row_specs(_BR_LO, 0) + [
            pl.BlockSpec((_BR_LO, _C), lambda i: (i, 0)),
            pl.BlockSpec((_BR_LO, _C), lambda i: (i, 0)),
            pl.BlockSpec((_BR_LO, _C), lambda i: (i, 0)),
            pl.BlockSpec(memory_space=pltpu.MemorySpace.HBM),
        ],
        out_specs=pl.BlockSpec((_BR_LO, _C), lambda i: (i, 0)),
        out_shape=jax.ShapeDtypeStruct((_B, _C), jnp.float32),
        input_output_aliases={12: 0},
        compiler_params=pltpu.CompilerParams(
            dimension_semantics=("parallel",)),
    )(input_0, input_1, input_2, sp3, av3, W0, W1, W2, bstack,
      u0, u1, u2, buf)


# speedup vs baseline: 1.0418x; 1.0418x over previous
import functools

import jax
import jax.numpy as jnp
import numpy as np
from jax import lax
from jax.experimental import pallas as pl
from jax.experimental.pallas import tpu as pltpu
from jax.experimental.pallas import tpu_sc as plsc

_B = 16384
_C = 256
_M = 3
_BR_HI = 1024
_BR_LO = 2048
_B_SC = 4096

_NC = 2
_NS = 16
_NW = _NC * _NS
_NW_ELEMS = _B_SC * _C // _NW

_TINY = np.float32(np.finfo(np.float32).tiny)
_SCALE = np.float32(1.0) - _TINY

_K0 = np.uint32(0)
_K1 = np.uint32(42)
_K2 = _K0 ^ _K1 ^ np.uint32(0x1BD11BDA)
_KS = (_K0, _K1, _K2)
_ROTS = ((13, 15, 26, 6), (17, 29, 16, 24))


def _threefry_bits(j):
    x0 = jnp.zeros_like(j) + _K0
    x1 = j + _K1
    for i in range(5):
        for r in _ROTS[i % 2]:
            x0 = x0 + x1
            x1 = (x1 << np.uint32(r)) | (x1 >> np.uint32(32 - r))
            x1 = x1 ^ x0
        x0 = x0 + _KS[(i + 1) % 3]
        x1 = x1 + _KS[(i + 2) % 3] + np.uint32(i + 1)
    return x0 ^ x1


def _uniform_from_bits(bits):
    fb = (bits >> np.uint32(9)) | np.uint32(0x3F800000)
    f = jax.lax.bitcast_convert_type(fb, jnp.float32) - np.float32(1.0)
    return f + _TINY


def _neglog_u_from_count(j):
    return -jnp.log(_uniform_from_bits(_threefry_bits(j)))


def _sc_noise_body(u0_hbm, u1_hbm, u2_hbm, v0, v1, v2):
    wid = lax.axis_index("s") * _NC + lax.axis_index("c")
    base = wid * _NW_ELEMS
    lane3 = lax.iota(jnp.int32, 16) * 3

    def body(i, carry):
        jv = ((base + i * 16) * 3 + lane3).astype(jnp.uint32)
        sl = pl.ds(i * 16, 16)
        v0[sl] = _uniform_from_bits(_threefry_bits(jv))
        v1[sl] = _uniform_from_bits(_threefry_bits(jv + np.uint32(1)))
        v2[sl] = _uniform_from_bits(_threefry_bits(jv + np.uint32(2)))
        return carry

    lax.fori_loop(0, _NW_ELEMS // 16, body, 0)
    pltpu.sync_copy(v0, u0_hbm.at[pl.ds(base, _NW_ELEMS)])
    pltpu.sync_copy(v1, u1_hbm.at[pl.ds(base, _NW_ELEMS)])
    pltpu.sync_copy(v2, u2_hbm.at[pl.ds(base, _NW_ELEMS)])


_sc_noise = functools.partial(
    pl.kernel,
    out_type=[jax.ShapeDtypeStruct((_B_SC * _C,), jnp.float32)] * 3,
    mesh=plsc.VectorSubcoreMesh(core_axis_name="c", subcore_axis_name="s"),
    scratch_types=[pltpu.VMEM((_NW_ELEMS,), jnp.float32)] * 3,
)(_sc_noise_body)


def _docking(x0_ref, x1_ref, x2_ref, w0_ref, w1_ref, w2_ref, b_ref):
    return (_dock(x0_ref, w0_ref, b_ref, 0),
            _dock(x1_ref, w1_ref, b_ref, 1),
            _dock(x2_ref, w2_ref, b_ref, 2))


def _select(p, l0, l1, l2, d0, d1, d2):
    p0 = p[:, 0:1]
    p1 = p[:, 1:2]
    p2 = p[:, 2:3]
    pick0 = (p0 * l1 >= p1 * l0) & (p0 * l2 >= p2 * l0)
    pick1 = p1 * l2 >= p2 * l1
    return jnp.where(pick0, d0, jnp.where(pick1, d1, d2))


def _probs(sp_ref, av_ref):
    w = sp_ref[...] * av_ref[...]
    return w / jnp.sum(w, axis=1, keepdims=True) + np.float32(1e-20)


def _dock(x_ref, w_ref, b_ref, row):
    return jnp.maximum(
        lax.dot_general(x_ref[...], w_ref[...],
                        (((1,), (1,)), ((), ())),
                        preferred_element_type=jnp.float32)
        + b_ref[row:row + 1, :], 0.0)


def _hi_kernel(x0_ref, x1_ref, x2_ref, sp_ref, av_ref,
               w0_ref, w1_ref, w2_ref, b_ref, out_ref):
    d0, d1, d2 = _docking(x0_ref, x1_ref, x2_ref, w0_ref, w1_ref, w2_ref,
                          b_ref)
    p = _probs(sp_ref, av_ref)
    base = (_B_SC + pl.program_id(0) * _BR_HI) * (_C * _M)
    r = jax.lax.broadcasted_iota(jnp.int32, (_BR_HI, _C), 0)
    c = jax.lax.broadcasted_iota(jnp.int32, (_BR_HI, _C), 1)
    q = base + r * (_C * _M) + c * _M
    l0 = _neglog_u_from_count(q.astype(jnp.uint32))
    l1 = _neglog_u_from_count((q + 1).astype(jnp.uint32))
    l2 = _neglog_u_from_count((q + 2).astype(jnp.uint32))
    out_ref[...] = _select(p, l0, l1, l2, d0, d1, d2)


def _lo_kernel(x0_ref, x1_ref, x2_ref, sp_ref, av_ref,
               w0_ref, w1_ref, w2_ref, b_ref,
               u0_ref, u1_ref, u2_ref, buf_ref, out_ref):
    del buf_ref
    d0, d1, d2 = _docking(x0_ref, x1_ref, x2_ref, w0_ref, w1_ref, w2_ref,
                          b_ref)
    p = _probs(sp_ref, av_ref)
    l0 = -jnp.log(u0_ref[...])
    l1 = -jnp.log(u1_ref[...])
    l2 = -jnp.log(u2_ref[...])
    out_ref[...] = _select(p, l0, l1, l2, d0, d1, d2)


def _row_specs(br, off):
    def rows(i, _off=off):
        return (i + _off, 0)

    return [
        pl.BlockSpec((br, 512), rows),
        pl.BlockSpec((br, 256), rows),
        pl.BlockSpec((br, 128), rows),
        pl.BlockSpec((br, 3), rows),
        pl.BlockSpec((br, 3), rows),
        pl.BlockSpec((256, 512), lambda i: (0, 0)),
        pl.BlockSpec((256, 256), lambda i: (0, 0)),
        pl.BlockSpec((256, 128), lambda i: (0, 0)),
        pl.BlockSpec((3, 256), lambda i: (0, 0)),
    ]


def kernel(input_0, input_1, input_2, availabilities, selection_probabilities,
           W0, b0, W1, b1, W2, b2):
    sp3 = selection_probabilities.astype(jnp.float32)
    av3 = availabilities.astype(jnp.float32)
    bstack = jnp.stack([b0, b1, b2], axis=0)

    u0, u1, u2 = _sc_noise()
    u0 = u0.reshape(_B_SC, _C)
    u1 = u1.reshape(_B_SC, _C)
    u2 = u2.reshape(_B_SC, _C)

    nsc = _B_SC // _BR_HI
    buf = pl.pallas_call(
        _hi_kernel,
        grid=((_B - _B_SC) // _BR_HI,),
        in_specs=_row_specs(_BR_HI, nsc),
        out_specs=pl.BlockSpec((_BR_HI, _C), lambda i, _off=nsc: (i + _off, 0)),
        out_shape=jax.ShapeDtypeStruct((_B, _C), jnp.float32),
        compiler_params=pltpu.CompilerParams(
            dimension_semantics=("parallel",)),
    )(input_0, input_1, input_2, sp3, av3, W0, W1, W2, bstack)

    return pl.pallas_call(
        _lo_kernel,
        grid=(_B_SC // _BR_LO,),
        in_specs=_row_specs(_BR_LO, 0) + [
            pl.BlockSpec((_BR_LO, _C), lambda i: (i, 0)),
            pl.BlockSpec((_BR_LO, _C), lambda i: (i, 0)),
            pl.BlockSpec((_BR_LO, _C), lambda i: (i, 0)),
            pl.BlockSpec(memory_space=pltpu.MemorySpace.HBM),
        ],
        out_specs=pl.BlockSpec((_BR_LO, _C), lambda i: (i, 0)),
        out_shape=jax.ShapeDtypeStruct((_B, _C), jnp.float32),
        input_output_aliases={12: 0},
        compiler_params=pltpu.CompilerParams(
            dimension_semantics=("parallel",)),
    )(input_0, input_1, input_2, sp3, av3, W0, W1, W2, bstack,
      u0, u1, u2, buf)


# speedup vs baseline: 1.0462x; 1.0043x over previous
import functools

import jax
import jax.numpy as jnp
import numpy as np
from jax import lax
from jax.experimental import pallas as pl
from jax.experimental.pallas import tpu as pltpu
from jax.experimental.pallas import tpu_sc as plsc

_B = 16384
_C = 256
_M = 3
_BR_HI = 1024
_BR_LO = 1024
_B_SC = 4096

_NC = 2
_NS = 16
_NW = _NC * _NS
_NW_ELEMS = _B_SC * _C // _NW

_TINY = np.float32(np.finfo(np.float32).tiny)
_SCALE = np.float32(1.0) - _TINY

_K0 = np.uint32(0)
_K1 = np.uint32(42)
_K2 = _K0 ^ _K1 ^ np.uint32(0x1BD11BDA)
_KS = (_K0, _K1, _K2)
_ROTS = ((13, 15, 26, 6), (17, 29, 16, 24))


def _threefry_bits(j):
    x0 = jnp.zeros_like(j) + _K0
    x1 = j + _K1
    for i in range(5):
        for r in _ROTS[i % 2]:
            x0 = x0 + x1
            x1 = (x1 << np.uint32(r)) | (x1 >> np.uint32(32 - r))
            x1 = x1 ^ x0
        x0 = x0 + _KS[(i + 1) % 3]
        x1 = x1 + _KS[(i + 2) % 3] + np.uint32(i + 1)
    return x0 ^ x1


def _uniform_from_bits(bits):
    fb = (bits >> np.uint32(9)) | np.uint32(0x3F800000)
    f = jax.lax.bitcast_convert_type(fb, jnp.float32) - np.float32(1.0)
    return f + _TINY


def _neglog_u_from_count(j):
    return -jnp.log(_uniform_from_bits(_threefry_bits(j)))


def _sc_noise_body(u0_hbm, u1_hbm, u2_hbm, v0, v1, v2):
    wid = lax.axis_index("s") * _NC + lax.axis_index("c")
    base = wid * _NW_ELEMS
    lane3 = lax.iota(jnp.int32, 16) * 3

    def body(i, carry):
        jv = ((base + i * 16) * 3 + lane3).astype(jnp.uint32)
        sl = pl.ds(i * 16, 16)
        v0[sl] = _uniform_from_bits(_threefry_bits(jv))
        v1[sl] = _uniform_from_bits(_threefry_bits(jv + np.uint32(1)))
        v2[sl] = _uniform_from_bits(_threefry_bits(jv + np.uint32(2)))
        return carry

    lax.fori_loop(0, _NW_ELEMS // 16, body, 0)
    pltpu.sync_copy(v0, u0_hbm.at[pl.ds(base, _NW_ELEMS)])
    pltpu.sync_copy(v1, u1_hbm.at[pl.ds(base, _NW_ELEMS)])
    pltpu.sync_copy(v2, u2_hbm.at[pl.ds(base, _NW_ELEMS)])


_sc_noise = functools.partial(
    pl.kernel,
    out_type=[jax.ShapeDtypeStruct((_B_SC * _C,), jnp.float32)] * 3,
    mesh=plsc.VectorSubcoreMesh(core_axis_name="c", subcore_axis_name="s"),
    scratch_types=[pltpu.VMEM((_NW_ELEMS,), jnp.float32)] * 3,
)(_sc_noise_body)


def _docking(x0_ref, x1_ref, x2_ref, w0_ref, w1_ref, w2_ref, b_ref):
    return (_dock(x0_ref, w0_ref, b_ref, 0),
            _dock(x1_ref, w1_ref, b_ref, 1),
            _dock(x2_ref, w2_ref, b_ref, 2))


def _select(p, l0, l1, l2, d0, d1, d2):
    p0 = p[:, 0:1]
    p1 = p[:, 1:2]
    p2 = p[:, 2:3]
    pick0 = (p0 * l1 >= p1 * l0) & (p0 * l2 >= p2 * l0)
    pick1 = p1 * l2 >= p2 * l1
    return jnp.where(pick0, d0, jnp.where(pick1, d1, d2))


def _probs(sp_ref, av_ref):
    w = sp_ref[...] * av_ref[...]
    return w / jnp.sum(w, axis=1, keepdims=True) + np.float32(1e-20)


def _dock(x_ref, w_ref, b_ref, row):
    return jnp.maximum(
        lax.dot_general(x_ref[...], w_ref[...],
                        (((1,), (1,)), ((), ())),
                        preferred_element_type=jnp.float32)
        + b_ref[row:row + 1, :], 0.0)


def _hi_kernel(x0_ref, x1_ref, x2_ref, sp_ref, av_ref,
               w0_ref, w1_ref, w2_ref, b_ref, out_ref):
    d0, d1, d2 = _docking(x0_ref, x1_ref, x2_ref, w0_ref, w1_ref, w2_ref,
                          b_ref)
    p = _probs(sp_ref, av_ref)
    base = (_B_SC + pl.program_id(0) * _BR_HI) * (_C * _M)
    r = jax.lax.broadcasted_iota(jnp.int32, (_BR_HI, _C), 0)
    c = jax.lax.broadcasted_iota(jnp.int32, (_BR_HI, _C), 1)
    q = base + r * (_C * _M) + c * _M
    l0 = _neglog_u_from_count(q.astype(jnp.uint32))
    l1 = _neglog_u_from_count((q + 1).astype(jnp.uint32))
    l2 = _neglog_u_from_count((q + 2).astype(jnp.uint32))
    out_ref[...] = _select(p, l0, l1, l2, d0, d1, d2)


def _lo_kernel(x0_ref, x1_ref, x2_ref, sp_ref, av_ref,
               w0_ref, w1_ref, w2_ref, b_ref,
               u0_ref, u1_ref, u2_ref, buf_ref, out_ref):
    del buf_ref
    d0, d1, d2 = _docking(x0_ref, x1_ref, x2_ref, w0_ref, w1_ref, w2_ref,
                          b_ref)
    p = _probs(sp_ref, av_ref)
    l0 = -jnp.log(u0_ref[...])
    l1 = -jnp.log(u1_ref[...])
    l2 = -jnp.log(u2_ref[...])
    out_ref[...] = _select(p, l0, l1, l2, d0, d1, d2)


def _row_specs(br, off):
    def rows(i, _off=off):
        return (i + _off, 0)

    return [
        pl.BlockSpec((br, 512), rows),
        pl.BlockSpec((br, 256), rows),
        pl.BlockSpec((br, 128), rows),
        pl.BlockSpec((br, 3), rows),
        pl.BlockSpec((br, 3), rows),
        pl.BlockSpec((256, 512), lambda i: (0, 0)),
        pl.BlockSpec((256, 256), lambda i: (0, 0)),
        pl.BlockSpec((256, 128), lambda i: (0, 0)),
        pl.BlockSpec((3, 256), lambda i: (0, 0)),
    ]


def kernel(input_0, input_1, input_2, availabilities, selection_probabilities,
           W0, b0, W1, b1, W2, b2):
    sp3 = selection_probabilities.astype(jnp.float32)
    av3 = availabilities.astype(jnp.float32)
    bstack = jnp.stack([b0, b1, b2], axis=0)

    u0, u1, u2 = _sc_noise()
    u0 = u0.reshape(_B_SC, _C)
    u1 = u1.reshape(_B_SC, _C)
    u2 = u2.reshape(_B_SC, _C)

    nsc = _B_SC // _BR_HI
    buf = pl.pallas_call(
        _hi_kernel,
        grid=((_B - _B_SC) // _BR_HI,),
        in_specs=_row_specs(_BR_HI, nsc),
        out_specs=pl.BlockSpec((_BR_HI, _C), lambda i, _off=nsc: (i + _off, 0)),
        out_shape=jax.ShapeDtypeStruct((_B, _C), jnp.float32),
        compiler_params=pltpu.CompilerParams(
            dimension_semantics=("parallel",)),
    )(input_0, input_1, input_2, sp3, av3, W0, W1, W2, bstack)

    return pl.pallas_call(
        _lo_kernel,
        grid=(_B_SC // _BR_LO,),
        in_specs=_row_specs(_BR_LO, 0) + [
            pl.BlockSpec((_BR_LO, _C), lambda i: (i, 0)),
            pl.BlockSpec((_BR_LO, _C), lambda i: (i, 0)),
            pl.BlockSpec((_BR_LO, _C), lambda i: (i, 0)),
            pl.BlockSpec(memory_space=pltpu.MemorySpace.HBM),
        ],
        out_specs=pl.BlockSpec((_BR_LO, _C), lambda i: (i, 0)),
        out_shape=jax.ShapeDtypeStruct((_B, _C), jnp.float32),
        input_output_aliases={12: 0},
        compiler_params=pltpu.CompilerParams(
            dimension_semantics=("parallel",)),
    )(input_0, input_1, input_2, sp3, av3, W0, W1, W2, bstack,
      u0, u1, u2, buf)
